# Initial kernel scaffold; baseline (speedup 1.0000x reference)
import jax
import jax.numpy as jnp
from jax.experimental import pallas as pl


def kernel(node_features, edge_index, edge_features, global_features, params):
    raise NotImplementedError("write your pallas kernel here")


# speedup vs baseline: 6.2364x; 6.2364x over previous
import functools
import jax
import jax.numpy as jnp
from jax import lax
from jax.experimental import pallas as pl
from jax.experimental.pallas import tpu as pltpu
from jax.experimental.pallas import tpu_sc as plsc

_N = 10000
_E = 320000
_NW = 32
_ET = _E // _NW
_C = 100
_NCH = _ET // _C
_ROWS = _E // _C
_RPT = 1000
_ZT = _N // _RPT


def _edge_sc(G, s2d, d2d, Pm, Qm, zer):
    mesh = plsc.VectorSubcoreMesh(core_axis_name="c", subcore_axis_name="s")

    @functools.partial(
        pl.kernel,
        mesh=mesh,
        compiler_params=pltpu.CompilerParams(use_tc_tiling_on_sc=False),
        out_type=[
            jax.ShapeDtypeStruct((_ROWS, _C, 16), jnp.float32),
            jax.ShapeDtypeStruct((2, _N, 32), jnp.float32),
        ],
        scratch_types=[
            pltpu.VMEM((_NCH, _C), jnp.int32),
            pltpu.VMEM((_NCH, _C), jnp.int32),
            pltpu.VMEM((_C, 32), jnp.float32),
            pltpu.VMEM((_C, 32), jnp.float32),
            pltpu.VMEM((_C, 16), jnp.float32),
            pltpu.VMEM((_C, 16), jnp.float32),
            pltpu.VMEM((_C, 32), jnp.float32),
            pltpu.VMEM((_C, 32), jnp.float32),
            pltpu.VMEM((_C, 16), jnp.float32),
            pltpu.VMEM_SHARED((_N, 32), jnp.float32),
            pltpu.SemaphoreType.DMA,
            pltpu.SemaphoreType.DMA,
        ],
    )
    def k(G_h, s_h, d_h, P_h, Q_h, z_h, eres_h, S2_h,
          sidx, didx, gsb, gdb, pb, qb, ub, vb, rb, sacc, sem1, sem2):
        cid = lax.axis_index("c")
        sid = lax.axis_index("s")
        wid = sid * 2 + cid
        pltpu.sync_copy(s_h.at[wid], sidx)
        pltpu.sync_copy(d_h.at[wid], didx)

        @pl.when(sid < _ZT)
        def _():
            pltpu.sync_copy(z_h, sacc.at[pl.ds(sid * _RPT, _RPT)])
        ones = jnp.ones((16,), jnp.float32)

        def initrow(r, carry):
            ub[r, pl.ds(16, 16)] = ones
            vb[r, pl.ds(16, 16)] = ones
            return carry

        lax.fori_loop(0, _C, initrow, 0)
        plsc.subcore_barrier()

        rbase = wid * _NCH

        def chunk(i, carry):
            pltpu.sync_copy(P_h.at[rbase + i], pb)
            pltpu.sync_copy(Q_h.at[rbase + i], qb)
            cp1 = pltpu.async_copy(G_h.at[sidx.at[i]], gsb, sem1)
            cp2 = pltpu.async_copy(G_h.at[didx.at[i]], gdb, sem2)
            cp1.wait()
            cp2.wait()

            def row(r, c2):
                pr = pb[r, :]
                ub[r, pl.ds(0, 16)] = gsb[r, pl.ds(0, 16)] + pr
                vb[r, pl.ds(0, 16)] = gdb[r, pl.ds(0, 16)] + pr
                rb[r, :] = (gsb[r, pl.ds(16, 16)] + gdb[r, pl.ds(16, 16)]) * 0.5 + qb[r, :]
                return c2

            lax.fori_loop(0, _C, row, 0)
            pltpu.sync_copy(rb, eres_h.at[rbase + i])
            pltpu.sync_copy(ub, sacc.at[didx.at[i]], add=True)
            pltpu.sync_copy(vb, sacc.at[sidx.at[i]], add=True)
            return carry

        lax.fori_loop(0, _NCH, chunk, 0)
        plsc.subcore_barrier()

        @pl.when(sid < _ZT)
        def _():
            pltpu.sync_copy(sacc.at[pl.ds(sid * _RPT, _RPT)],
                            S2_h.at[cid, pl.ds(sid * _RPT, _RPT)])

    return k(G, s2d, d2d, Pm, Qm, zer)


def _node_table(x, W48):
    def body(x_ref, w_ref, o_ref):
        o_ref[...] = jnp.dot(x_ref[...], w_ref[...],
                             preferred_element_type=jnp.float32)

    return pl.pallas_call(
        body,
        grid=(5,),
        in_specs=[pl.BlockSpec((2000, 128), lambda i: (i, 0)),
                  pl.BlockSpec((128, 48), lambda i: (0, 0))],
        out_specs=pl.BlockSpec((2000, 48), lambda i: (i, 0)),
        out_shape=jax.ShapeDtypeStruct((_N, 48), jnp.float32),
    )(x, W48)


def _pq(ea, Wr, r0, bede):
    BE = 8000

    def body(ea_ref, wr_ref, r0_ref, bd_ref, p_ref, q_ref):
        blk = ea_ref[...]
        pv = jnp.dot(blk, wr_ref[...], preferred_element_type=jnp.float32) + r0_ref[...]
        p_ref[...] = pv
        q_ref[...] = pv + bd_ref[...] + blk

    return pl.pallas_call(
        body,
        grid=(_E // BE,),
        in_specs=[pl.BlockSpec((BE, 16), lambda i: (i, 0)),
                  pl.BlockSpec((16, 16), lambda i: (0, 0)),
                  pl.BlockSpec((1, 16), lambda i: (0, 0)),
                  pl.BlockSpec((1, 16), lambda i: (0, 0))],
        out_specs=[pl.BlockSpec((BE, 16), lambda i: (i, 0)),
                   pl.BlockSpec((BE, 16), lambda i: (i, 0))],
        out_shape=[jax.ShapeDtypeStruct((_E, 16), jnp.float32),
                   jax.ShapeDtypeStruct((_E, 16), jnp.float32)],
    )(ea, Wr, r0, bede)


def _node_update(x, m, W1, W2, cvec, Wnd, bnd):
    def body(x_ref, m_ref, w1_ref, w2_ref, cv_ref, wnd_ref, bd_ref,
             o_ref, ns_ref):
        i = pl.program_id(0)
        xb = x_ref[...]
        pre = jnp.dot(xb, w1_ref[...], preferred_element_type=jnp.float32)
        pre = pre + jnp.dot(m_ref[...], w2_ref[...],
                            preferred_element_type=jnp.float32)
        pre = pre + cv_ref[...]
        pre = jnp.dot(pre, wnd_ref[...],
                      preferred_element_type=jnp.float32) + bd_ref[...]
        o_ref[...] = pre + xb

        @pl.when(i == 0)
        def _():
            ns_ref[...] = jnp.zeros_like(ns_ref)

        ns_ref[...] += jnp.sum(pre, axis=0, keepdims=True)

    return pl.pallas_call(
        body,
        grid=(5,),
        in_specs=[pl.BlockSpec((2000, 128), lambda i: (i, 0)),
                  pl.BlockSpec((2000, 16), lambda i: (i, 0)),
                  pl.BlockSpec((128, 32), lambda i: (0, 0)),
                  pl.BlockSpec((16, 32), lambda i: (0, 0)),
                  pl.BlockSpec((1, 32), lambda i: (0, 0)),
                  pl.BlockSpec((32, 128), lambda i: (0, 0)),
                  pl.BlockSpec((1, 128), lambda i: (0, 0))],
        out_specs=[pl.BlockSpec((2000, 128), lambda i: (i, 0)),
                   pl.BlockSpec((1, 128), lambda i: (0, 0))],
        out_shape=[jax.ShapeDtypeStruct((_N, 128), jnp.float32),
                   jax.ShapeDtypeStruct((1, 128), jnp.float32)],
    )(x, m, W1, W2, cvec, Wnd, bnd)


def _s2s_pass(xm, q, B):
    M, d = xm.shape

    def body(x_ref, q_ref, o_ref, m_sc, s_sc, r_acc):
        i = pl.program_id(0)

        @pl.when(i == 0)
        def _():
            m_sc[0, 0] = -1e30
            s_sc[0, 0] = 0.0
            r_acc[...] = jnp.zeros_like(r_acc)

        blk = x_ref[...]
        scv = jnp.sum(blk * q_ref[...], axis=1, keepdims=True)
        bm = jnp.max(scv)
        m_old = m_sc[0, 0]
        m_new = jnp.maximum(m_old, bm)
        scale = jnp.exp(m_old - m_new)
        pvec = jnp.exp(scv - m_new)
        s_sc[0, 0] = s_sc[0, 0] * scale + jnp.sum(pvec)
        r_acc[...] = r_acc[...] * scale + jnp.sum(pvec * blk, axis=0,
                                                  keepdims=True)
        m_sc[0, 0] = m_new

        @pl.when(i == pl.num_programs(0) - 1)
        def _():
            o_ref[...] = r_acc[...] / s_sc[0, 0]

    return pl.pallas_call(
        body,
        grid=(M // B,),
        in_specs=[pl.BlockSpec((B, d), lambda i: (i, 0)),
                  pl.BlockSpec((1, d), lambda i: (0, 0))],
        out_specs=pl.BlockSpec((1, d), lambda i: (0, 0)),
        out_shape=jax.ShapeDtypeStruct((1, d), jnp.float32),
        scratch_shapes=[pltpu.SMEM((1, 1), jnp.float32),
                        pltpu.SMEM((1, 1), jnp.float32),
                        pltpu.VMEM((1, d), jnp.float32)],
    )(xm, q)


def _set2set(x, Wih, Whh, bih, bhh, B):
    d = x.shape[1]
    q_star = jnp.zeros((1, 2 * d), x.dtype)
    h = jnp.zeros((1, d), x.dtype)
    c = jnp.zeros((1, d), x.dtype)
    for _ in range(3):
        gates = q_star @ Wih + bih + h @ Whh + bhh
        ig, fg, gg, og = jnp.split(gates, 4, axis=-1)
        c = jax.nn.sigmoid(fg) * c + jax.nn.sigmoid(ig) * jnp.tanh(gg)
        h = jax.nn.sigmoid(og) * jnp.tanh(c)
        r = _s2s_pass(x, h, B)
        q_star = jnp.concatenate([h, r], axis=-1)
    return q_star


def kernel(node_features, edge_index, edge_features, global_features, params):
    x = node_features
    ea = edge_features
    g = global_features
    p = params
    We1, Wed = p['We1'], p['Wed']

    A16 = We1[:128] @ Wed
    B16 = We1[128:256] @ Wed
    W48 = jnp.concatenate([A16, A16 + B16, B16], axis=1)
    XW = _node_table(x, W48)
    G = XW[:, :32]
    xbw = XW[:, 32:48]

    r0 = (g @ We1[272:304] + p['be1']) @ Wed
    Pm, Qm = _pq(ea, We1[256:272] @ Wed, r0, p['bed'][None, :])

    s2d = edge_index[0].reshape(_NW, _NCH, _C)
    d2d = edge_index[1].reshape(_NW, _NCH, _C)
    zer = jnp.zeros((_RPT, 32), jnp.float32)
    eres3, S2 = _edge_sc(G, s2d, d2d,
                         Pm.reshape(_ROWS, _C, 16), Qm.reshape(_ROWS, _C, 16),
                         zer)
    eres = eres3.reshape(_E, 16)

    Ssum = S2[0] + S2[1]
    cnt = Ssum[:, 16:17]
    Swf = Ssum[:, :16] + cnt * xbw
    m = Swf / jnp.maximum(cnt, 1.0) + p['bed'] * (cnt > 0)
    e_mean = jnp.sum(Swf, axis=0, keepdims=True) / (2 * _E) + p['bed']

    Wn1 = p['Wn1']
    cvec = g @ Wn1[144:176] + p['bn1'][None, :]
    n_new, nsum = _node_update(x, m, Wn1[:128], Wn1[128:144], cvec,
                               p['Wnd'], p['bnd'][None, :])
    n_mean = nsum / _N

    g_in = jnp.concatenate([e_mean, n_mean, g], axis=1)
    g_new = (g_in @ p['Wg1'] + p['bg1']) @ p['Wgd'] + p['bgd'] + g

    s2s_n = _set2set(n_new, p['Wih_n'], p['Whh_n'], p['bih_n'], p['bhh_n'], 2000)
    s2s_e = _set2set(eres, p['Wih_e'], p['Whh_e'], p['bih_e'], p['bhh_e'], 8000)

    out = jnp.concatenate([s2s_n[0], s2s_e[0], g_new[0]], axis=0)
    out = out @ p['Wd1'] + p['bd1']
    out = out @ p['Wd2'] + p['bd2']
    return out @ p['Wout'] + p['bout']


# speedup vs baseline: 8.9800x; 1.4399x over previous
import functools
import jax
import jax.numpy as jnp
from jax import lax
from jax.experimental import pallas as pl
from jax.experimental.pallas import tpu as pltpu
from jax.experimental.pallas import tpu_sc as plsc

_N = 10000
_E = 320000
_NW = 32
_ET = _E // _NW
_C = 100
_NCH = _ET // _C
_ROWS = _E // _C
_RPT = 1000
_ZT = _N // _RPT


def _edge_sc(G, s2d, d2d, PQ, zer):
    mesh = plsc.VectorSubcoreMesh(core_axis_name="c", subcore_axis_name="s")

    @functools.partial(
        pl.kernel,
        mesh=mesh,
        compiler_params=pltpu.CompilerParams(use_tc_tiling_on_sc=False),
        out_type=[
            jax.ShapeDtypeStruct((_ROWS, _C, 16), jnp.float32),
            jax.ShapeDtypeStruct((2, _N, 32), jnp.float32),
        ],
        scratch_types=[
            pltpu.VMEM((_NCH, _C), jnp.int32),
            pltpu.VMEM((_NCH, _C), jnp.int32),
            pltpu.VMEM((2, _C, 32), jnp.float32),
            pltpu.VMEM((2, _C, 32), jnp.float32),
            pltpu.VMEM((2, _C, 32), jnp.float32),
            pltpu.VMEM((2, _C, 32), jnp.float32),
            pltpu.VMEM((2, _C, 32), jnp.float32),
            pltpu.VMEM((2, _C, 16), jnp.float32),
            pltpu.VMEM_SHARED((_N, 32), jnp.float32),
            pltpu.SemaphoreType.DMA,
            pltpu.SemaphoreType.DMA,
            pltpu.SemaphoreType.DMA,
            pltpu.SemaphoreType.DMA,
        ],
    )
    def k(G_h, s_h, d_h, PQ_h, z_h, eres_h, S2_h,
          sidx, didx, gsb, gdb, pqb, ub, vb, rb, sacc,
          semg, semp, semo, semo2):
        cid = lax.axis_index("c")
        sid = lax.axis_index("s")
        wid = sid * 2 + cid
        pltpu.sync_copy(s_h.at[wid], sidx)
        pltpu.sync_copy(d_h.at[wid], didx)

        @pl.when(sid < _ZT)
        def _():
            pltpu.sync_copy(z_h, sacc.at[pl.ds(sid * _RPT, _RPT)])

        ones = jnp.ones((16,), jnp.float32)

        def initrow(r, carry):
            ub[0, r, pl.ds(16, 16)] = ones
            ub[1, r, pl.ds(16, 16)] = ones
            vb[0, r, pl.ds(16, 16)] = ones
            vb[1, r, pl.ds(16, 16)] = ones
            return carry

        lax.fori_loop(0, _C, initrow, 0)
        plsc.subcore_barrier()

        rbase = wid * _NCH

        def in_copies(i, b):
            return (pltpu.make_async_copy(PQ_h.at[rbase + i], pqb.at[b], semp),
                    pltpu.make_async_copy(G_h.at[sidx.at[i]], gsb.at[b], semg),
                    pltpu.make_async_copy(G_h.at[didx.at[i]], gdb.at[b], semg))

        def out_copies(i, b):
            return (pltpu.make_async_copy(rb.at[b], eres_h.at[rbase + i], semo),
                    pltpu.make_async_copy(ub.at[b], sacc.at[didx.at[i]], semo2),
                    pltpu.make_async_copy(vb.at[b], sacc.at[sidx.at[i]], semo2))

        def start_out(i, b):
            cps = out_copies(i, b)
            cps[0].start()
            cps[1].start(add=True)
            cps[2].start(add=True)

        for cp in in_copies(0, 0) + in_copies(1, 1):
            cp.start()

        def pair(g, carry):
            for b in (0, 1):
                i = 2 * g + b
                for cp in in_copies(i, b):
                    cp.wait()

                @pl.when(g > 0)
                def _():
                    for cp in out_copies(i - 2, b):
                        cp.wait()

                def row(r, c2):
                    pr = pqb[b, r, pl.ds(0, 16)]
                    ub[b, r, pl.ds(0, 16)] = gsb[b, r, pl.ds(0, 16)] + pr
                    vb[b, r, pl.ds(0, 16)] = gdb[b, r, pl.ds(0, 16)] + pr
                    rb[b, r, :] = ((gsb[b, r, pl.ds(16, 16)]
                                    + gdb[b, r, pl.ds(16, 16)]) * 0.5
                                   + pqb[b, r, pl.ds(16, 16)])
                    return c2

                lax.fori_loop(0, _C, row, 0)
                start_out(i, b)

                @pl.when(g + 1 < _NCH // 2)
                def _():
                    for cp in in_copies(i + 2, b):
                        cp.start()

            return carry

        lax.fori_loop(0, _NCH // 2, pair, 0)
        for b in (0, 1):
            for cp in out_copies(_NCH - 2 + b, b):
                cp.wait()
        plsc.subcore_barrier()

        @pl.when(sid < _ZT)
        def _():
            pltpu.sync_copy(sacc.at[pl.ds(sid * _RPT, _RPT)],
                            S2_h.at[cid, pl.ds(sid * _RPT, _RPT)])

    return k(G, s2d, d2d, PQ, zer)


def _node_table(x, W48):
    def body(x_ref, w_ref, o_ref):
        o_ref[...] = jnp.dot(x_ref[...], w_ref[...],
                             preferred_element_type=jnp.float32)

    return pl.pallas_call(
        body,
        grid=(5,),
        in_specs=[pl.BlockSpec((2000, 128), lambda i: (i, 0)),
                  pl.BlockSpec((128, 48), lambda i: (0, 0))],
        out_specs=pl.BlockSpec((2000, 48), lambda i: (i, 0)),
        out_shape=jax.ShapeDtypeStruct((_N, 48), jnp.float32),
    )(x, W48)


def _pq(ea, Wr, r0, bede):
    BE = 8000

    def body(ea_ref, wr_ref, r0_ref, bd_ref, pq_ref):
        blk = ea_ref[...]
        pv = jnp.dot(blk, wr_ref[...], preferred_element_type=jnp.float32) + r0_ref[...]
        pq_ref[...] = jnp.concatenate([pv, pv + bd_ref[...] + blk], axis=1)

    return pl.pallas_call(
        body,
        grid=(_E // BE,),
        in_specs=[pl.BlockSpec((BE, 16), lambda i: (i, 0)),
                  pl.BlockSpec((16, 16), lambda i: (0, 0)),
                  pl.BlockSpec((1, 16), lambda i: (0, 0)),
                  pl.BlockSpec((1, 16), lambda i: (0, 0))],
        out_specs=pl.BlockSpec((BE, 32), lambda i: (i, 0)),
        out_shape=jax.ShapeDtypeStruct((_E, 32), jnp.float32),
    )(ea, Wr, r0, bede)


def _node_update(x, m, W1, W2, cvec, Wnd, bnd):
    def body(x_ref, m_ref, w1_ref, w2_ref, cv_ref, wnd_ref, bd_ref,
             o_ref, ns_ref):
        i = pl.program_id(0)
        xb = x_ref[...]
        pre = jnp.dot(xb, w1_ref[...], preferred_element_type=jnp.float32)
        pre = pre + jnp.dot(m_ref[...], w2_ref[...],
                            preferred_element_type=jnp.float32)
        pre = pre + cv_ref[...]
        pre = jnp.dot(pre, wnd_ref[...],
                      preferred_element_type=jnp.float32) + bd_ref[...]
        o_ref[...] = pre + xb

        @pl.when(i == 0)
        def _():
            ns_ref[...] = jnp.zeros_like(ns_ref)

        ns_ref[...] += jnp.sum(pre, axis=0, keepdims=True)

    return pl.pallas_call(
        body,
        grid=(5,),
        in_specs=[pl.BlockSpec((2000, 128), lambda i: (i, 0)),
                  pl.BlockSpec((2000, 16), lambda i: (i, 0)),
                  pl.BlockSpec((128, 32), lambda i: (0, 0)),
                  pl.BlockSpec((16, 32), lambda i: (0, 0)),
                  pl.BlockSpec((1, 32), lambda i: (0, 0)),
                  pl.BlockSpec((32, 128), lambda i: (0, 0)),
                  pl.BlockSpec((1, 128), lambda i: (0, 0))],
        out_specs=[pl.BlockSpec((2000, 128), lambda i: (i, 0)),
                   pl.BlockSpec((1, 128), lambda i: (0, 0))],
        out_shape=[jax.ShapeDtypeStruct((_N, 128), jnp.float32),
                   jax.ShapeDtypeStruct((1, 128), jnp.float32)],
    )(x, m, W1, W2, cvec, Wnd, bnd)


def _s2s_pass(xm, q, B):
    M, d = xm.shape

    def body(x_ref, q_ref, o_ref, m_sc, s_sc, r_acc):
        i = pl.program_id(0)

        @pl.when(i == 0)
        def _():
            m_sc[0, 0] = -1e30
            s_sc[0, 0] = 0.0
            r_acc[...] = jnp.zeros_like(r_acc)

        blk = x_ref[...]
        scv = jnp.sum(blk * q_ref[...], axis=1, keepdims=True)
        bm = jnp.max(scv)
        m_old = m_sc[0, 0]
        m_new = jnp.maximum(m_old, bm)
        scale = jnp.exp(m_old - m_new)
        pvec = jnp.exp(scv - m_new)
        s_sc[0, 0] = s_sc[0, 0] * scale + jnp.sum(pvec)
        r_acc[...] = r_acc[...] * scale + jnp.sum(pvec * blk, axis=0,
                                                  keepdims=True)
        m_sc[0, 0] = m_new

        @pl.when(i == pl.num_programs(0) - 1)
        def _():
            o_ref[...] = r_acc[...] / s_sc[0, 0]

    return pl.pallas_call(
        body,
        grid=(M // B,),
        in_specs=[pl.BlockSpec((B, d), lambda i: (i, 0)),
                  pl.BlockSpec((1, d), lambda i: (0, 0))],
        out_specs=pl.BlockSpec((1, d), lambda i: (0, 0)),
        out_shape=jax.ShapeDtypeStruct((1, d), jnp.float32),
        scratch_shapes=[pltpu.SMEM((1, 1), jnp.float32),
                        pltpu.SMEM((1, 1), jnp.float32),
                        pltpu.VMEM((1, d), jnp.float32)],
    )(xm, q)


def _set2set(x, Wih, Whh, bih, bhh, B):
    d = x.shape[1]
    q_star = jnp.zeros((1, 2 * d), x.dtype)
    h = jnp.zeros((1, d), x.dtype)
    c = jnp.zeros((1, d), x.dtype)
    for _ in range(3):
        gates = q_star @ Wih + bih + h @ Whh + bhh
        ig, fg, gg, og = jnp.split(gates, 4, axis=-1)
        c = jax.nn.sigmoid(fg) * c + jax.nn.sigmoid(ig) * jnp.tanh(gg)
        h = jax.nn.sigmoid(og) * jnp.tanh(c)
        r = _s2s_pass(x, h, B)
        q_star = jnp.concatenate([h, r], axis=-1)
    return q_star


def kernel(node_features, edge_index, edge_features, global_features, params):
    x = node_features
    ea = edge_features
    g = global_features
    p = params
    We1, Wed = p['We1'], p['Wed']

    A16 = We1[:128] @ Wed
    B16 = We1[128:256] @ Wed
    W48 = jnp.concatenate([A16, A16 + B16, B16], axis=1)
    XW = _node_table(x, W48)
    G = XW[:, :32]
    xbw = XW[:, 32:48]

    r0 = (g @ We1[272:304] + p['be1']) @ Wed
    PQm = _pq(ea, We1[256:272] @ Wed, r0, p['bed'][None, :])

    s2d = edge_index[0].reshape(_NW, _NCH, _C)
    d2d = edge_index[1].reshape(_NW, _NCH, _C)
    zer = jnp.zeros((_RPT, 32), jnp.float32)
    eres3, S2 = _edge_sc(G, s2d, d2d, PQm.reshape(_ROWS, _C, 32), zer)
    eres = eres3.reshape(_E, 16)

    Ssum = S2[0] + S2[1]
    cnt = Ssum[:, 16:17]
    Swf = Ssum[:, :16] + cnt * xbw
    m = Swf / jnp.maximum(cnt, 1.0) + p['bed'] * (cnt > 0)
    e_mean = jnp.sum(Swf, axis=0, keepdims=True) / (2 * _E) + p['bed']

    Wn1 = p['Wn1']
    cvec = g @ Wn1[144:176] + p['bn1'][None, :]
    n_new, nsum = _node_update(x, m, Wn1[:128], Wn1[128:144], cvec,
                               p['Wnd'], p['bnd'][None, :])
    n_mean = nsum / _N

    g_in = jnp.concatenate([e_mean, n_mean, g], axis=1)
    g_new = (g_in @ p['Wg1'] + p['bg1']) @ p['Wgd'] + p['bgd'] + g

    s2s_n = _set2set(n_new, p['Wih_n'], p['Whh_n'], p['bih_n'], p['bhh_n'], 2000)
    s2s_e = _set2set(eres, p['Wih_e'], p['Whh_e'], p['bih_e'], p['bhh_e'], 8000)

    out = jnp.concatenate([s2s_n[0], s2s_e[0], g_new[0]], axis=0)
    out = out @ p['Wd1'] + p['bd1']
    out = out @ p['Wd2'] + p['bd2']
    return out @ p['Wout'] + p['bout']


# speedup vs baseline: 9.2618x; 1.0314x over previous
import functools
import jax
import jax.numpy as jnp
from jax import lax
from jax.experimental import pallas as pl
from jax.experimental.pallas import tpu as pltpu
from jax.experimental.pallas import tpu_sc as plsc

_N = 10000
_E = 320000
_NW = 32
_ET = _E // _NW
_C = 100
_NCH = _ET // _C
_ROWS = _E // _C
_RPT = 1000
_ZT = _N // _RPT


def _edge_sc(G, s2d, d2d, PQ, zer):
    mesh = plsc.VectorSubcoreMesh(core_axis_name="c", subcore_axis_name="s")

    @functools.partial(
        pl.kernel,
        mesh=mesh,
        compiler_params=pltpu.CompilerParams(use_tc_tiling_on_sc=False),
        out_type=[
            jax.ShapeDtypeStruct((_ROWS, _C, 16), jnp.float32),
            jax.ShapeDtypeStruct((2, _N, 32), jnp.float32),
        ],
        scratch_types=[
            pltpu.VMEM((_NCH, _C), jnp.int32),
            pltpu.VMEM((_NCH, _C), jnp.int32),
            pltpu.VMEM((2, _C, 32), jnp.float32),
            pltpu.VMEM((2, _C, 32), jnp.float32),
            pltpu.VMEM((2, _C, 32), jnp.float32),
            pltpu.VMEM((2, _C, 32), jnp.float32),
            pltpu.VMEM((2, _C, 32), jnp.float32),
            pltpu.VMEM((2, _C, 16), jnp.float32),
            pltpu.VMEM_SHARED((_N, 32), jnp.float32),
            pltpu.SemaphoreType.DMA,
            pltpu.SemaphoreType.DMA,
            pltpu.SemaphoreType.DMA,
            pltpu.SemaphoreType.DMA,
        ],
    )
    def k(G_h, s_h, d_h, PQ_h, z_h, eres_h, S2_h,
          sidx, didx, gsb, gdb, pqb, ub, vb, rb, sacc,
          semg, semp, semo, semo2):
        cid = lax.axis_index("c")
        sid = lax.axis_index("s")
        wid = sid * 2 + cid
        pltpu.sync_copy(s_h.at[wid], sidx)
        pltpu.sync_copy(d_h.at[wid], didx)

        @pl.when(sid < _ZT)
        def _():
            pltpu.sync_copy(z_h, sacc.at[pl.ds(sid * _RPT, _RPT)])

        ones = jnp.ones((16,), jnp.float32)

        def initrow(r, carry):
            ub[0, r, pl.ds(16, 16)] = ones
            ub[1, r, pl.ds(16, 16)] = ones
            vb[0, r, pl.ds(16, 16)] = ones
            vb[1, r, pl.ds(16, 16)] = ones
            return carry

        lax.fori_loop(0, _C, initrow, 0)
        plsc.subcore_barrier()

        rbase = wid * _NCH

        def in_copies(i, b):
            return (pltpu.make_async_copy(PQ_h.at[rbase + i], pqb.at[b], semp),
                    pltpu.make_async_copy(G_h.at[sidx.at[i]], gsb.at[b], semg),
                    pltpu.make_async_copy(G_h.at[didx.at[i]], gdb.at[b], semg))

        def out_copies(i, b):
            return (pltpu.make_async_copy(rb.at[b], eres_h.at[rbase + i], semo),
                    pltpu.make_async_copy(ub.at[b], sacc.at[didx.at[i]], semo2),
                    pltpu.make_async_copy(vb.at[b], sacc.at[sidx.at[i]], semo2))

        def start_out(i, b):
            cps = out_copies(i, b)
            cps[0].start()
            cps[1].start(add=True)
            cps[2].start(add=True)

        for cp in in_copies(0, 0) + in_copies(1, 1):
            cp.start()

        def pair(g, carry):
            for b in (0, 1):
                i = 2 * g + b
                for cp in in_copies(i, b):
                    cp.wait()

                @pl.when(g > 0)
                def _():
                    for cp in out_copies(i - 2, b):
                        cp.wait()

                def row(r, c2):
                    pr = pqb[b, r, pl.ds(0, 16)]
                    ub[b, r, pl.ds(0, 16)] = gsb[b, r, pl.ds(0, 16)] + pr
                    vb[b, r, pl.ds(0, 16)] = gdb[b, r, pl.ds(0, 16)] + pr
                    rb[b, r, :] = ((gsb[b, r, pl.ds(16, 16)]
                                    + gdb[b, r, pl.ds(16, 16)]) * 0.5
                                   + pqb[b, r, pl.ds(16, 16)])
                    return c2

                lax.fori_loop(0, _C, row, 0)
                start_out(i, b)

                @pl.when(g + 1 < _NCH // 2)
                def _():
                    for cp in in_copies(i + 2, b):
                        cp.start()

            return carry

        lax.fori_loop(0, _NCH // 2, pair, 0)
        for b in (0, 1):
            for cp in out_copies(_NCH - 2 + b, b):
                cp.wait()
        plsc.subcore_barrier()

        @pl.when(sid < _ZT)
        def _():
            pltpu.sync_copy(sacc.at[pl.ds(sid * _RPT, _RPT)],
                            S2_h.at[cid, pl.ds(sid * _RPT, _RPT)])

    return k(G, s2d, d2d, PQ, zer)


def _node_table(x, W48):
    def body(x_ref, w_ref, o_ref):
        o_ref[...] = jnp.dot(x_ref[...], w_ref[...],
                             preferred_element_type=jnp.float32)

    return pl.pallas_call(
        body,
        grid=(5,),
        in_specs=[pl.BlockSpec((2000, 128), lambda i: (i, 0)),
                  pl.BlockSpec((128, 48), lambda i: (0, 0))],
        out_specs=pl.BlockSpec((2000, 48), lambda i: (i, 0)),
        out_shape=jax.ShapeDtypeStruct((_N, 48), jnp.float32),
    )(x, W48)


def _pq(ea, Wr, r0, bede):
    BE = 8000

    def body(ea_ref, wr_ref, r0_ref, bd_ref, pq_ref):
        blk = ea_ref[...]
        pv = jnp.dot(blk, wr_ref[...], preferred_element_type=jnp.float32) + r0_ref[...]
        pq_ref[...] = jnp.concatenate([pv, pv + bd_ref[...] + blk], axis=1)

    return pl.pallas_call(
        body,
        grid=(_E // BE,),
        in_specs=[pl.BlockSpec((BE, 16), lambda i: (i, 0)),
                  pl.BlockSpec((16, 16), lambda i: (0, 0)),
                  pl.BlockSpec((1, 16), lambda i: (0, 0)),
                  pl.BlockSpec((1, 16), lambda i: (0, 0))],
        out_specs=pl.BlockSpec((BE, 32), lambda i: (i, 0)),
        out_shape=jax.ShapeDtypeStruct((_E, 32), jnp.float32),
    )(ea, Wr, r0, bede)


def _node_update(x, S2, XW, bed, W1, W2, cvec, Wnd, bnd):
    B = 2000

    def body(x_ref, s2_ref, xw_ref, bed_ref, w1_ref, w2_ref, cv_ref,
             wnd_ref, bd_ref, o_ref, ns_ref, sw_ref):
        i = pl.program_id(0)
        xb = x_ref[...]
        s2 = s2_ref[...]
        ss = s2[0] + s2[1]
        cnt = ss[:, 16:17]
        xbw = xw_ref[:, 32:48]
        swf = ss[:, 0:16] + cnt * xbw
        mloc = swf / jnp.maximum(cnt, 1.0) + bed_ref[...] * (cnt > 0)
        pre = jnp.dot(xb, w1_ref[...], preferred_element_type=jnp.float32)
        pre = pre + jnp.dot(mloc, w2_ref[...],
                            preferred_element_type=jnp.float32)
        pre = pre + cv_ref[...]
        pre = jnp.dot(pre, wnd_ref[...],
                      preferred_element_type=jnp.float32) + bd_ref[...]
        o_ref[...] = pre + xb

        @pl.when(i == 0)
        def _():
            ns_ref[...] = jnp.zeros_like(ns_ref)
            sw_ref[...] = jnp.zeros_like(sw_ref)

        ns_ref[...] += jnp.sum(pre, axis=0, keepdims=True)
        sw_ref[...] += jnp.sum(swf, axis=0, keepdims=True)

    return pl.pallas_call(
        body,
        grid=(_N // B,),
        in_specs=[pl.BlockSpec((B, 128), lambda i: (i, 0)),
                  pl.BlockSpec((2, B, 32), lambda i: (0, i, 0)),
                  pl.BlockSpec((B, 48), lambda i: (i, 0)),
                  pl.BlockSpec((1, 16), lambda i: (0, 0)),
                  pl.BlockSpec((128, 32), lambda i: (0, 0)),
                  pl.BlockSpec((16, 32), lambda i: (0, 0)),
                  pl.BlockSpec((1, 32), lambda i: (0, 0)),
                  pl.BlockSpec((32, 128), lambda i: (0, 0)),
                  pl.BlockSpec((1, 128), lambda i: (0, 0))],
        out_specs=[pl.BlockSpec((B, 128), lambda i: (i, 0)),
                   pl.BlockSpec((1, 128), lambda i: (0, 0)),
                   pl.BlockSpec((1, 16), lambda i: (0, 0))],
        out_shape=[jax.ShapeDtypeStruct((_N, 128), jnp.float32),
                   jax.ShapeDtypeStruct((1, 128), jnp.float32),
                   jax.ShapeDtypeStruct((1, 16), jnp.float32)],
    )(x, S2, XW, bed, W1, W2, cvec, Wnd, bnd)


def _set2set(x, Wih, Whh, bih, bhh, B):
    M, d = x.shape
    NB = M // B
    Wa = Wih[:d] + Whh
    Wb = Wih[d:]
    bias = (bih + bhh)[None, :]

    def body(x_ref, wa_ref, wb_ref, b_ref, o_ref,
             h_sc, c_sc, r_sc, m_sc, s_sc, r_acc):
        t = pl.program_id(0)
        i = pl.program_id(1)

        @pl.when(i == 0)
        def _():
            @pl.when(t == 0)
            def _():
                h_sc[...] = jnp.zeros_like(h_sc)
                c_sc[...] = jnp.zeros_like(c_sc)
                r_sc[...] = jnp.zeros_like(r_sc)

            gates = (jnp.dot(h_sc[...], wa_ref[...],
                             preferred_element_type=jnp.float32)
                     + jnp.dot(r_sc[...], wb_ref[...],
                               preferred_element_type=jnp.float32)
                     + b_ref[...])
            ig = gates[:, 0:d]
            fg = gates[:, d:2 * d]
            gg = gates[:, 2 * d:3 * d]
            og = gates[:, 3 * d:4 * d]
            cv = (jax.nn.sigmoid(fg) * c_sc[...]
                  + jax.nn.sigmoid(ig) * jnp.tanh(gg))
            h_sc[...] = jax.nn.sigmoid(og) * jnp.tanh(cv)
            c_sc[...] = cv
            m_sc[0, 0] = -1e30
            s_sc[0, 0] = 0.0
            r_acc[...] = jnp.zeros_like(r_acc)

        blk = x_ref[...]
        scv = jnp.sum(blk * h_sc[...], axis=1, keepdims=True)
        bm = jnp.max(scv)
        m_old = m_sc[0, 0]
        m_new = jnp.maximum(m_old, bm)
        scale = jnp.exp(m_old - m_new)
        pvec = jnp.exp(scv - m_new)
        s_sc[0, 0] = s_sc[0, 0] * scale + jnp.sum(pvec)
        r_acc[...] = r_acc[...] * scale + jnp.sum(pvec * blk, axis=0,
                                                  keepdims=True)
        m_sc[0, 0] = m_new

        @pl.when(i == NB - 1)
        def _():
            r_sc[...] = r_acc[...] / s_sc[0, 0]

            @pl.when(t == 2)
            def _():
                o_ref[0, :] = h_sc[0, :]
                o_ref[1, :] = r_sc[0, :]

    out = pl.pallas_call(
        body,
        grid=(3, NB),
        in_specs=[pl.BlockSpec((B, d), lambda t, i: (i, 0)),
                  pl.BlockSpec((d, 4 * d), lambda t, i: (0, 0)),
                  pl.BlockSpec((d, 4 * d), lambda t, i: (0, 0)),
                  pl.BlockSpec((1, 4 * d), lambda t, i: (0, 0))],
        out_specs=pl.BlockSpec((2, d), lambda t, i: (0, 0)),
        out_shape=jax.ShapeDtypeStruct((2, d), jnp.float32),
        scratch_shapes=[pltpu.VMEM((1, d), jnp.float32),
                        pltpu.VMEM((1, d), jnp.float32),
                        pltpu.VMEM((1, d), jnp.float32),
                        pltpu.SMEM((1, 1), jnp.float32),
                        pltpu.SMEM((1, 1), jnp.float32),
                        pltpu.VMEM((1, d), jnp.float32)],
    )(x, Wa, Wb, bias)
    return jnp.concatenate([out[0], out[1]], axis=0)[None, :]


def kernel(node_features, edge_index, edge_features, global_features, params):
    x = node_features
    ea = edge_features
    g = global_features
    p = params
    We1, Wed = p['We1'], p['Wed']

    A16 = We1[:128] @ Wed
    B16 = We1[128:256] @ Wed
    W48 = jnp.concatenate([A16, A16 + B16, B16], axis=1)
    XW = _node_table(x, W48)
    G = XW[:, :32]
    xbw = XW[:, 32:48]

    r0 = (g @ We1[272:304] + p['be1']) @ Wed
    PQm = _pq(ea, We1[256:272] @ Wed, r0, p['bed'][None, :])

    s2d = edge_index[0].reshape(_NW, _NCH, _C)
    d2d = edge_index[1].reshape(_NW, _NCH, _C)
    zer = jnp.zeros((_RPT, 32), jnp.float32)
    eres3, S2 = _edge_sc(G, s2d, d2d, PQm.reshape(_ROWS, _C, 32), zer)
    eres = eres3.reshape(_E, 16)

    Wn1 = p['Wn1']
    cvec = g @ Wn1[144:176] + p['bn1'][None, :]
    n_new, nsum, swsum = _node_update(x, S2, XW, p['bed'][None, :],
                                      Wn1[:128], Wn1[128:144], cvec,
                                      p['Wnd'], p['bnd'][None, :])
    e_mean = swsum / (2 * _E) + p['bed']
    n_mean = nsum / _N

    g_in = jnp.concatenate([e_mean, n_mean, g], axis=1)
    g_new = (g_in @ p['Wg1'] + p['bg1']) @ p['Wgd'] + p['bgd'] + g

    s2s_n = _set2set(n_new, p['Wih_n'], p['Whh_n'], p['bih_n'], p['bhh_n'], 2000)
    s2s_e = _set2set(eres, p['Wih_e'], p['Whh_e'], p['bih_e'], p['bhh_e'], 8000)

    out = jnp.concatenate([s2s_n[0], s2s_e[0], g_new[0]], axis=0)
    out = out @ p['Wd1'] + p['bd1']
    out = out @ p['Wd2'] + p['bd2']
    return out @ p['Wout'] + p['bout']
